# baseline (speedup 1.0000x reference)
import functools

import jax
import jax.numpy as jnp
from jax import lax
from jax.experimental import pallas as pl
from jax.experimental.pallas import tpu as pltpu
from jax.experimental.pallas import tpu_sc as plsc

NUM_ROWS = 100000
DIM = 64
CHUNK = 128
NUM_FULL = NUM_ROWS // CHUNK
TAIL = NUM_ROWS - NUM_FULL * CHUNK
NUM_WORKERS = 32
STEPS = (NUM_FULL + NUM_WORKERS - 1) // NUM_WORKERS


def _make_gather():
    mesh = plsc.VectorSubcoreMesh(core_axis_name="c", subcore_axis_name="s")

    @functools.partial(
        pl.kernel,
        mesh=mesh,
        out_type=jax.ShapeDtypeStruct((NUM_ROWS, DIM), jnp.float32),
        scratch_types=[
            pltpu.VMEM((CHUNK,), jnp.int32),
            pltpu.VMEM((CHUNK, DIM), jnp.float32),
            pltpu.VMEM((TAIL,), jnp.int32),
            pltpu.VMEM((TAIL, DIM), jnp.float32),
            pltpu.SemaphoreType.DMA,
        ],
        compiler_params=pltpu.CompilerParams(use_tc_tiling_on_sc=False),
    )
    def gather_kernel(w_hbm, idx_hbm, out_hbm, idx_v, rows_v, idx_t, rows_t, sem):
        wid = lax.axis_index("s") * 2 + lax.axis_index("c")

        def step(t, carry):
            j = wid + t * NUM_WORKERS

            @pl.when(j < NUM_FULL)
            def _():
                base = j * CHUNK
                pltpu.sync_copy(idx_hbm.at[pl.ds(base, CHUNK)], idx_v)
                pltpu.async_copy(w_hbm.at[idx_v], rows_v, sem).wait()
                pltpu.sync_copy(rows_v, out_hbm.at[pl.ds(base, CHUNK)])

            return carry

        lax.fori_loop(0, STEPS, step, 0)

        @pl.when(wid == NUM_WORKERS - 1)
        def _():
            base = NUM_FULL * CHUNK
            pltpu.sync_copy(idx_hbm.at[pl.ds(base, TAIL)], idx_t)
            pltpu.async_copy(w_hbm.at[idx_t], rows_t, sem).wait()
            pltpu.sync_copy(rows_t, out_hbm.at[pl.ds(base, TAIL)])

    return gather_kernel


_gather = _make_gather()


@jax.jit
def kernel(W, indices):
    return _gather(W, indices)


# speedup vs baseline: 1.1782x; 1.1782x over previous
import functools

import jax
import jax.numpy as jnp
from jax import lax
from jax.experimental import pallas as pl
from jax.experimental.pallas import tpu as pltpu
from jax.experimental.pallas import tpu_sc as plsc

NUM_ROWS = 100000
DIM = 64
CHUNK = 128
NBUF = 5
NUM_WORKERS = 32
ROWS_PER_W = 3200
CHUNKS_PER_W = ROWS_PER_W // CHUNK
MAIN_ITERS = CHUNKS_PER_W // NBUF - 1
LAST_BASE = (NUM_WORKERS - 1) * ROWS_PER_W
LAST_ROWS = NUM_ROWS - LAST_BASE
LAST_FULL = LAST_ROWS // CHUNK
TAIL = LAST_ROWS - LAST_FULL * CHUNK


def _make_gather():
    mesh = plsc.VectorSubcoreMesh(core_axis_name="c", subcore_axis_name="s")

    @functools.partial(
        pl.kernel,
        mesh=mesh,
        out_type=jax.ShapeDtypeStruct((NUM_ROWS, DIM), jnp.float32),
        scratch_types=[
            pltpu.VMEM((ROWS_PER_W,), jnp.int32),
            pltpu.VMEM((NBUF * CHUNK, DIM), jnp.float32),
            pltpu.SemaphoreType.DMA((NBUF,)),
            pltpu.SemaphoreType.DMA((NBUF,)),
        ],
        compiler_params=pltpu.CompilerParams(use_tc_tiling_on_sc=False),
    )
    def gather_kernel(w_hbm, idx_hbm, out_hbm, idx_v, rows_v, gsem, ssem):
        wid = lax.axis_index("s") * 2 + lax.axis_index("c")
        base = wid * ROWS_PER_W

        def buf(b):
            return rows_v.at[pl.ds(b * CHUNK, CHUNK)]

        def idx_slice(i, n=CHUNK):
            return idx_v.at[pl.ds(i * CHUNK, n)]

        def gather(i, b):
            return pltpu.make_async_copy(w_hbm.at[idx_slice(i)], buf(b), gsem.at[b])

        def store(i, b):
            return pltpu.make_async_copy(
                buf(b), out_hbm.at[pl.ds(base + i * CHUNK, CHUNK)], ssem.at[b]
            )

        @pl.when(wid < NUM_WORKERS - 1)
        def _full_block():
            pltpu.sync_copy(idx_hbm.at[pl.ds(base, ROWS_PER_W)], idx_v)
            for b in range(NBUF):
                gather(b, b).start()

            def ring(t, carry):
                for b in range(NBUF):
                    i = t * NBUF + b
                    gather(i, b).wait()
                    store(i, b).start()
                for b in range(NBUF):
                    i = t * NBUF + b
                    store(i, b).wait()
                    gather(i + NBUF, b).start()
                return carry

            lax.fori_loop(0, MAIN_ITERS, ring, 0)

            t_last = MAIN_ITERS
            for b in range(NBUF):
                i = t_last * NBUF + b
                gather(i, b).wait()
                store(i, b).start()
            for b in range(NBUF):
                store(t_last * NBUF + b, b).wait()

        @pl.when(wid == NUM_WORKERS - 1)
        def _last_block():
            pltpu.sync_copy(
                idx_hbm.at[pl.ds(LAST_BASE, LAST_ROWS)], idx_v.at[pl.ds(0, LAST_ROWS)]
            )
            for c in range(LAST_FULL):
                b = c % NBUF
                gather(c, b).start()
                gather(c, b).wait()
                store(c, b).start()
                store(c, b).wait()
            tail_src = w_hbm.at[idx_v.at[pl.ds(LAST_FULL * CHUNK, TAIL)]]
            tail_buf = rows_v.at[pl.ds(0, TAIL)]
            pltpu.make_async_copy(tail_src, tail_buf, gsem.at[0]).start()
            pltpu.make_async_copy(tail_src, tail_buf, gsem.at[0]).wait()
            pltpu.sync_copy(
                tail_buf, out_hbm.at[pl.ds(LAST_BASE + LAST_FULL * CHUNK, TAIL)]
            )

    return gather_kernel


_gather = _make_gather()


@jax.jit
def kernel(W, indices):
    return _gather(W, indices)
